# baseline (speedup 1.0000x reference)
import functools

import jax
import jax.numpy as jnp
from jax import lax
from jax.experimental import pallas as pl
from jax.experimental.pallas import tpu as pltpu
from jax.experimental.pallas import tpu_sc as plsc

_BATCH = 16384
_FEAT = 64
_NC = 2
_NS = 16
_NW = _NC * _NS
_BPW = _BATCH // _NW
_CHUNK = 128
_NCHUNK = _BPW // _CHUNK
_LANES = 16

_mesh = plsc.VectorSubcoreMesh(core_axis_name="c", subcore_axis_name="s")


@functools.partial(
    pl.kernel,
    mesh=_mesh,
    out_type=jax.ShapeDtypeStruct((_NW, _LANES), jnp.float32),
    scratch_types=[
        pltpu.VMEM((_NCHUNK, _CHUNK), jnp.int32),
        pltpu.VMEM((2, _CHUNK, 2 * _FEAT), jnp.float32),
        pltpu.VMEM((_BPW, _FEAT), jnp.float32),
        pltpu.VMEM((_LANES,), jnp.float32),
        pltpu.VMEM((_BPW,), jnp.int32),
        pltpu.SemaphoreType.DMA,
        pltpu.SemaphoreType.DMA,
        pltpu.SemaphoreType.DMA,
    ],
)
def _center_loss_sc(feat_hbm, lab_hbm, cent2_hbm, out_hbm,
                    idx_v, cen_v, feat_v, acc_v, off_v, gsem0, gsem1, fsem):
    wid = lax.axis_index("s") * _NC + lax.axis_index("c")
    base = wid * _BPW
    gsems = (gsem0, gsem1)

    fcopy = pltpu.async_copy(feat_hbm.at[pl.ds(base, _BPW)], feat_v, fsem)

    for j in range(_NCHUNK):
        pltpu.sync_copy(lab_hbm.at[pl.ds(base + j * _CHUNK, _CHUNK)],
                        idx_v.at[j])

    for j in range(_NCHUNK):
        for k in range(_CHUNK // _LANES):
            sl = pl.ds(k * _LANES, _LANES)
            lab = idx_v[j, sl]
            off_v[pl.ds(j * _CHUNK + k * _LANES, _LANES)] = (lab & 1) * _FEAT
            idx_v[j, sl] = lab >> 1

    def gather(c):
        return pltpu.async_copy(cent2_hbm.at[idx_v.at[c]],
                                cen_v.at[c % 2], gsems[c % 2])

    copies = [gather(0)]
    acc = jnp.zeros((_LANES,), jnp.float32)
    for c in range(_NCHUNK):
        if c + 1 < _NCHUNK:
            copies.append(gather(c + 1))
        copies[c].wait()
        if c == 0:
            fcopy.wait()

        def body(g, a, c=c):
            offs = off_v[pl.ds(c * _CHUNK + g * _LANES, _LANES)]
            for k in range(_LANES):
                r = g * _LANES + k
                i = c * _CHUNK + r
                off = offs[k]
                for j in range(_FEAT // _LANES):
                    d = (feat_v[i, pl.ds(j * _LANES, _LANES)]
                         - cen_v[c % 2, r, pl.ds(off + j * _LANES, _LANES)])
                    a = a + d * d
            return a

        acc = lax.fori_loop(0, _CHUNK // _LANES, body, acc)

    acc_v[...] = acc * (1.0 / (2.0 * _BATCH))
    pltpu.sync_copy(acc_v, out_hbm.at[wid])


def kernel(features, labels, centers):
    centers2 = centers.reshape(centers.shape[0] // 2, 2 * _FEAT)
    partials = _center_loss_sc(features, labels.astype(jnp.int32), centers2)
    return jnp.sum(partials)


# speedup vs baseline: 1.6347x; 1.6347x over previous
import functools

import jax
import jax.numpy as jnp
from jax import lax
from jax.experimental import pallas as pl
from jax.experimental.pallas import tpu as pltpu
from jax.experimental.pallas import tpu_sc as plsc

_BATCH = 16384
_FEAT = 64
_NC = 2
_NS = 16
_NW = _NC * _NS
_BPW = _BATCH // _NW
_CHUNK = 64
_NCHUNK = _BPW // _CHUNK
_LANES = 16

_mesh = plsc.VectorSubcoreMesh(core_axis_name="c", subcore_axis_name="s")


@functools.partial(
    pl.kernel,
    mesh=_mesh,
    out_type=jax.ShapeDtypeStruct((_NW, _LANES), jnp.float32),
    scratch_types=[
        pltpu.VMEM((_NCHUNK, _CHUNK), jnp.int32),
        pltpu.VMEM((2, _CHUNK, _FEAT), jnp.float32),
        pltpu.VMEM((_BPW, _FEAT), jnp.float32),
        pltpu.VMEM((_LANES,), jnp.float32),
        pltpu.SemaphoreType.DMA,
        pltpu.SemaphoreType.DMA,
        pltpu.SemaphoreType.DMA,
    ],
)
def _center_loss_sc(feat_hbm, lab_hbm, cent_hbm, out_hbm,
                    idx_v, cen_v, feat_v, acc_v, gsem0, gsem1, fsem):
    wid = lax.axis_index("s") * _NC + lax.axis_index("c")
    base = wid * _BPW
    gsems = (gsem0, gsem1)

    fcopy = pltpu.async_copy(feat_hbm.at[pl.ds(base, _BPW)], feat_v, fsem)

    for j in range(_NCHUNK):
        pltpu.sync_copy(lab_hbm.at[pl.ds(base + j * _CHUNK, _CHUNK)],
                        idx_v.at[j])

    def fetch(c):
        handles = []
        for v in range(_CHUNK // _LANES):
            rows = idx_v[c, pl.ds(v * _LANES, _LANES)]
            for k in range(_LANES):
                s = v * _LANES + k
                handles.append(pltpu.async_copy(
                    cent_hbm.at[pl.ds(rows[k], 1)],
                    cen_v.at[c % 2, pl.ds(s, 1)],
                    gsems[c % 2]))
        return handles

    copies = [fetch(0)]
    acc = jnp.zeros((_LANES,), jnp.float32)
    for c in range(_NCHUNK):
        if c + 1 < _NCHUNK:
            copies.append(fetch(c + 1))
        for h in copies[c]:
            h.wait()
        if c == 0:
            fcopy.wait()

        def body(g, a, c=c):
            for k in range(_LANES):
                s = g * _LANES + k
                i = c * _CHUNK + s
                for j in range(_FEAT // _LANES):
                    d = (feat_v[i, pl.ds(j * _LANES, _LANES)]
                         - cen_v[c % 2, s, pl.ds(j * _LANES, _LANES)])
                    a = a + d * d
            return a

        acc = lax.fori_loop(0, _CHUNK // _LANES, body, acc)

    acc_v[...] = acc * (1.0 / (2.0 * _BATCH))
    pltpu.sync_copy(acc_v, out_hbm.at[wid])


def kernel(features, labels, centers):
    partials = _center_loss_sc(features, labels.astype(jnp.int32), centers)
    return jnp.sum(partials)
